# baseline (speedup 1.0000x reference)
import functools

import jax
import jax.numpy as jnp
from jax import lax
from jax.experimental import pallas as pl
from jax.experimental.pallas import tpu as pltpu
from jax.experimental.pallas import tpu_sc as plsc

_N = 10000
_D = 128
_NPAD = 10240
_NC = 2
_NS = 16
_NW = _NC * _NS
_CHUNK = 128
_E2 = 320000 + _N
_CT = -(-_E2 // (_NW * _CHUNK))
_E2PAD = _NW * _CT * _CHUNK
_CHB = 256
_CTB = -(-_E2 // (_NW * _CHB))
_EB = _NW * _CTB * _CHB
_E2RD = _EB
_NEG = -1e30


def _vfill(ref, n16, val, dtype):
  v = jnp.full((16,), val, dtype)

  def body(i, _):
    ref[pl.ds(pl.multiple_of(i * 16, 16), 16)] = v
    return 0

  lax.fori_loop(0, n16, body, 0)


def _take(x, idx):
  dn = lax.GatherDimensionNumbers(offset_dims=(), collapsed_slice_dims=(0,),
                                  start_index_map=(0,))
  return lax.gather(x, idx[:, None], dn, slice_sizes=(1,),
                    mode=lax.GatherScatterMode.PROMISE_IN_BOUNDS)


def _seg_reduce_rmw(dstv, vals, acc_ref, combine):
  sk, sv = plsc.sort_key_val(dstv, vals)
  io = lax.iota(jnp.int32, 16)
  for off in (1, 2, 4, 8):
    idx = jnp.maximum(io - off, 0)
    same = (_take(sk, idx) == sk) & (io >= off)
    sv = jnp.where(same, combine(sv, _take(sv, idx)), sv)
  nk = _take(sk, jnp.minimum(io + 1, 15))
  is_last = (sk != nk) | (io == 15)
  old = plsc.load_gather(acc_ref, [sk], mask=is_last)
  plsc.store_scatter(acc_ref, [sk], combine(old, sv), mask=is_last)


def _tc_front_body(x_ref, g_ref, b_ref, wl_ref, bl_ref, wr_ref, br_ref,
                   xl_ref, xr_ref):
  x = x_ref[...]
  mu = jnp.mean(x, axis=0, keepdims=True)
  var = jnp.mean((x - mu) * (x - mu), axis=0, keepdims=True)
  h = (x - mu) * jax.lax.rsqrt(var + 1e-5) * g_ref[...][None, :] + b_ref[...][None, :]
  xl_ref[...] = jnp.dot(h, wl_ref[...], preferred_element_type=jnp.float32) + bl_ref[...][None, :]
  xr_ref[...] = jnp.dot(h, wr_ref[...], preferred_element_type=jnp.float32) + br_ref[...][None, :]


def _tc_front(x, g, b, wl, bl, wr, br):
  return pl.pallas_call(
      _tc_front_body,
      out_shape=[jax.ShapeDtypeStruct((_N, _D), jnp.float32),
                 jax.ShapeDtypeStruct((_N, _D), jnp.float32)],
  )(x, g, b, wl, bl, wr, br)


def _tc_mid_body(op_ref, dp_ref, bias_ref, g_ref, b_ref, wl_ref, bl_ref,
                 wr_ref, br_ref, xl_ref, xr_ref):
  num = op_ref[0, :_N, :] + op_ref[1, :_N, :]
  den = jnp.sum(dp_ref[...], axis=0)[: _N]
  out = num / den[:, None] + bias_ref[...][None, :]
  h = jnp.maximum(out, 0.0)
  mu = jnp.mean(h, axis=0, keepdims=True)
  var = jnp.mean((h - mu) * (h - mu), axis=0, keepdims=True)
  h = (h - mu) * jax.lax.rsqrt(var + 1e-5) * g_ref[...][None, :] + b_ref[...][None, :]
  xl_ref[...] = jnp.dot(h, wl_ref[...], preferred_element_type=jnp.float32) + bl_ref[...][None, :]
  xr_ref[...] = jnp.dot(h, wr_ref[...], preferred_element_type=jnp.float32) + br_ref[...][None, :]


def _tc_mid(op, dp, bias, g, b, wl, bl, wr, br):
  return pl.pallas_call(
      _tc_mid_body,
      out_shape=[jax.ShapeDtypeStruct((_N, _D), jnp.float32),
                 jax.ShapeDtypeStruct((_N, _D), jnp.float32)],
  )(op, dp, bias, g, b, wl, bl, wr, br)


def _tc_final_body(op_ref, dp_ref, bias_ref, out_ref):
  num = op_ref[0, :_N, :] + op_ref[1, :_N, :]
  den = jnp.sum(dp_ref[...], axis=0)[: _N]
  out_ref[...] = num / den[:, None] + bias_ref[...][None, :]


def _tc_final(op, dp, bias):
  return pl.pallas_call(
      _tc_final_body,
      out_shape=jax.ShapeDtypeStruct((_N, _D), jnp.float32),
  )(op, dp, bias)


def _sc_logits_body(xl_hbm, xr_hbm, att_hbm, sd_hbm,
                    logits_hbm, mparts_hbm,
                    att_v, sdidx, xlr, xrr, lg, m_local, sem0, sem1):
  cid = lax.axis_index("c")
  sid = lax.axis_index("s")
  wid = sid * _NC + cid
  pltpu.sync_copy(att_hbm, att_v)
  _vfill(m_local, _NPAD // 16, _NEG, jnp.float32)

  def chunk_body(c, _):
    base = pl.multiple_of((wid * _CTB + c) * _CHB, _CHB)
    pltpu.sync_copy(sd_hbm.at[wid * _CTB + c], sdidx)
    cps = [
        pltpu.async_copy(xl_hbm.at[sdidx.at[0]], xlr.at[pl.ds(0, 128)], sem0),
        pltpu.async_copy(xl_hbm.at[sdidx.at[1]], xlr.at[pl.ds(128, 128)], sem0),
        pltpu.async_copy(xr_hbm.at[sdidx.at[2]], xrr.at[pl.ds(0, 128)], sem1),
        pltpu.async_copy(xr_hbm.at[sdidx.at[3]], xrr.at[pl.ds(128, 128)], sem1),
    ]
    for cp in cps:
      cp.wait()

    io = lax.iota(jnp.int32, 16)
    for g in range(16):
      gx = pl.ds(g * 16, 16)

      def ebody(jj, acc, g=g):
        e = g * 16 + jj
        parts = []
        for j in range(8):
          jx = pl.ds(j * 16, 16)
          sval = xlr[e, jx] + xrr[e, jx]
          sval = jnp.maximum(sval, sval * 0.2)
          parts.append(sval * att_v[jx])
        while len(parts) > 1:
          parts = [a + b for a, b in zip(parts[::2], parts[1::2])]
        tot = jnp.sum(parts[0])
        return jnp.where(io == jj, tot, acc)

      lgv = lax.fori_loop(0, 16, ebody, jnp.zeros((16,), jnp.float32),
                          unroll=4)
      lg[gx] = lgv
      dstv = sdidx[2 + g // 8, pl.ds((g % 8) * 16, 16)]
      _seg_reduce_rmw(dstv, lgv, m_local, jnp.maximum)
    pltpu.sync_copy(lg, logits_hbm.at[pl.ds(base, _CHB)])
    return 0

  lax.fori_loop(0, _CTB, chunk_body, 0)
  pltpu.sync_copy(m_local, mparts_hbm.at[wid])


def _sc_logits(xl, xr, att, sd):
  mesh = plsc.VectorSubcoreMesh(core_axis_name="c", subcore_axis_name="s")
  f = pl.kernel(
      _sc_logits_body,
      compiler_params=pltpu.CompilerParams(needs_layout_passes=False),
      out_type=[jax.ShapeDtypeStruct((_EB,), jnp.float32),
                jax.ShapeDtypeStruct((_NW, _NPAD), jnp.float32)],
      mesh=mesh,
      scratch_types=[
          pltpu.VMEM((_D,), jnp.float32),
          pltpu.VMEM((4, _CHUNK), jnp.int32),
          pltpu.VMEM((_CHB, _D), jnp.float32),
          pltpu.VMEM((_CHB, _D), jnp.float32),
          pltpu.VMEM((_CHB,), jnp.float32),
          pltpu.VMEM((_NPAD,), jnp.float32),
          pltpu.SemaphoreType.DMA,
          pltpu.SemaphoreType.DMA,
      ],
  )
  return f(xl, xr, att, sd)


def _sc_aggregate_body(xl_hbm, sd_hbm, logits_hbm, mparts_hbm,
                       oparts_hbm, dparts_hbm,
                       sdidx, lg, rows, m_v, den_local, mstage, mloc,
                       shared, m_shared, sem0):
  cid = lax.axis_index("c")
  sid = lax.axis_index("s")
  wid = sid * _NC + cid
  stripe = _NPAD // _NS

  sbase = pl.multiple_of(sid * stripe, stripe)
  for r in range(2):
    cps = [pltpu.async_copy(mparts_hbm.at[r * _NS + t, pl.ds(sbase, stripe)],
                            mstage.at[t], sem0) for t in range(_NS)]
    for cp in cps:
      cp.wait()

    def cb(i, _, r=r):
      ix = pl.ds(pl.multiple_of(i * 16, 16), 16)
      parts = [mstage[t, ix] for t in range(_NS)]
      while len(parts) > 1:
        parts = [jnp.maximum(a, b) for a, b in zip(parts[::2], parts[1::2])]
      if r == 0:
        mloc[ix] = parts[0]
      else:
        mloc[ix] = jnp.maximum(mloc[ix], parts[0])
      return 0

    lax.fori_loop(0, stripe // 16, cb, 0)
  pltpu.sync_copy(mloc, m_shared.at[pl.ds(sbase, stripe)])

  _vfill(den_local, _NPAD // 16, 0.0, jnp.float32)

  def zrow(r, _):
    for j in range(8):
      rows[r, pl.ds(j * 16, 16)] = jnp.zeros((16,), jnp.float32)
    return 0

  lax.fori_loop(0, _CHUNK, zrow, 0)
  for k in range(stripe // _CHUNK):
    pltpu.sync_copy(rows, shared.at[pl.ds(sid * stripe + k * _CHUNK, _CHUNK)])
  plsc.subcore_barrier()
  pltpu.sync_copy(m_shared, m_v)

  def chunk_body(c, _):
    base = pl.multiple_of((wid * _CT + c) * _CHUNK, _CHUNK)
    pltpu.sync_copy(sd_hbm.at[wid * _CT + c], sdidx)
    pltpu.sync_copy(logits_hbm.at[pl.ds(base, _CHUNK)], lg)
    cp0 = pltpu.async_copy(xl_hbm.at[sdidx.at[0]], rows, sem0)
    cp0.wait()
    for g in range(8):
      gx = pl.ds(g * 16, 16)
      dstv = sdidx[1, gx]
      mg = plsc.load_gather(m_v, [dstv])
      w = jnp.exp(lg[gx] - mg)
      _seg_reduce_rmw(dstv, w, den_local, lax.add)
      for jj in range(16):
        e = g * 16 + jj
        ws = w[jj]
        for j in range(8):
          jx = pl.ds(j * 16, 16)
          rows[e, jx] = rows[e, jx] * ws

    pltpu.sync_copy(rows, shared.at[sdidx.at[1]], add=True)
    return 0

  lax.fori_loop(0, _CT, chunk_body, 0)
  plsc.subcore_barrier()
  pltpu.sync_copy(shared.at[pl.ds(sid * stripe, stripe)],
                  oparts_hbm.at[cid, pl.ds(sid * stripe, stripe)])
  pltpu.sync_copy(den_local, dparts_hbm.at[wid])


def _sc_aggregate(xl, sd, logits, mparts):
  mesh = plsc.VectorSubcoreMesh(core_axis_name="c", subcore_axis_name="s")
  f = pl.kernel(
      _sc_aggregate_body,
      compiler_params=pltpu.CompilerParams(needs_layout_passes=False),
      out_type=[jax.ShapeDtypeStruct((_NC, _NPAD, _D), jnp.float32),
                jax.ShapeDtypeStruct((_NW, _NPAD), jnp.float32)],
      mesh=mesh,
      scratch_types=[
          pltpu.VMEM((2, _CHUNK), jnp.int32),
          pltpu.VMEM((_CHUNK,), jnp.float32),
          pltpu.VMEM((_CHUNK, _D), jnp.float32),
          pltpu.VMEM((_NPAD,), jnp.float32),
          pltpu.VMEM((_NPAD,), jnp.float32),
          pltpu.VMEM((_NS, _NPAD // _NS), jnp.float32),
          pltpu.VMEM((_NPAD // _NS,), jnp.float32),
          pltpu.VMEM_SHARED((_NPAD, _D), jnp.float32),
          pltpu.VMEM_SHARED((_NPAD,), jnp.float32),
          pltpu.SemaphoreType.DMA,
      ],
  )
  return f(xl, sd, logits, mparts)


def _edge_stage(xl, xr, att, sdb, sdd):
  xlp = jnp.pad(xl, ((0, _NPAD - _N), (0, 0)))
  xrp = jnp.pad(xr, ((0, _NPAD - _N), (0, 0)))
  logits, mparts = _sc_logits(xlp, xrp, att.reshape(_D), sdb)
  return _sc_aggregate(xlp, sdd, logits, mparts)


def kernel(x, edge_index, edge_weight, bn1_g, bn1_b, Wl1, bl1, Wr1, br1,
           att1, bias1, bn2_g, bn2_b, Wl2, bl2, Wr2, br2, att2, bias2):
  del edge_weight
  src = edge_index[0]
  dst = edge_index[1]
  loop = jnp.arange(_N, dtype=src.dtype)
  pad = jnp.full((_E2RD - _E2,), _NPAD - 1, src.dtype)
  s2 = jnp.concatenate([src, loop, pad])
  d2 = jnp.concatenate([dst, loop, pad])
  sdb = jnp.concatenate([s2.reshape(-1, 2, _CHUNK), d2.reshape(-1, 2, _CHUNK)],
                        axis=1)
  sdd = jnp.stack([s2[:_E2PAD].reshape(-1, _CHUNK),
                   d2[:_E2PAD].reshape(-1, _CHUNK)], axis=1)

  xl1, xr1 = _tc_front(x, bn1_g, bn1_b, Wl1, bl1, Wr1, br1)
  op1, dp1 = _edge_stage(xl1, xr1, att1, sdb, sdd)
  xl2, xr2 = _tc_mid(op1, dp1, bias1, bn2_g, bn2_b, Wl2, bl2, Wr2, br2)
  op2, dp2 = _edge_stage(xl2, xr2, att2, sdb, sdd)
  return _tc_final(op2, dp2, bias2)


# speedup vs baseline: 1.2199x; 1.2199x over previous
import functools

import jax
import jax.numpy as jnp
from jax import lax
from jax.experimental import pallas as pl
from jax.experimental.pallas import tpu as pltpu
from jax.experimental.pallas import tpu_sc as plsc

_N = 10000
_D = 128
_NPAD = 10240
_NC = 2
_NS = 16
_NW = _NC * _NS
_CHUNK = 128
_E2 = 320000 + _N
_CT = -(-_E2 // (_NW * _CHUNK))
_E2PAD = _NW * _CT * _CHUNK
_E2RD = _E2PAD
_NEG = -1e30


def _vfill(ref, n16, val, dtype):
  v = jnp.full((16,), val, dtype)

  def body(i, _):
    ref[pl.ds(pl.multiple_of(i * 16, 16), 16)] = v
    return 0

  lax.fori_loop(0, n16, body, 0)


def _take(x, idx):
  dn = lax.GatherDimensionNumbers(offset_dims=(), collapsed_slice_dims=(0,),
                                  start_index_map=(0,))
  return lax.gather(x, idx[:, None], dn, slice_sizes=(1,),
                    mode=lax.GatherScatterMode.PROMISE_IN_BOUNDS)


def _seg_reduce_rmw(dstv, vals, acc_ref, combine):
  sk, sv = plsc.sort_key_val(dstv, vals)
  io = lax.iota(jnp.int32, 16)
  for off in (1, 2, 4, 8):
    idx = jnp.maximum(io - off, 0)
    same = (_take(sk, idx) == sk) & (io >= off)
    sv = jnp.where(same, combine(sv, _take(sv, idx)), sv)
  nk = _take(sk, jnp.minimum(io + 1, 15))
  is_last = (sk != nk) | (io == 15)
  old = plsc.load_gather(acc_ref, [sk], mask=is_last)
  plsc.store_scatter(acc_ref, [sk], combine(old, sv), mask=is_last)


def _tc_front_body(x_ref, g_ref, b_ref, wl_ref, bl_ref, wr_ref, br_ref,
                   xl_ref, xr_ref):
  x = x_ref[...]
  mu = jnp.mean(x, axis=0, keepdims=True)
  var = jnp.mean((x - mu) * (x - mu), axis=0, keepdims=True)
  h = (x - mu) * jax.lax.rsqrt(var + 1e-5) * g_ref[...][None, :] + b_ref[...][None, :]
  xl_ref[...] = jnp.dot(h, wl_ref[...], preferred_element_type=jnp.float32) + bl_ref[...][None, :]
  xr_ref[...] = jnp.dot(h, wr_ref[...], preferred_element_type=jnp.float32) + br_ref[...][None, :]


def _tc_front(x, g, b, wl, bl, wr, br):
  return pl.pallas_call(
      _tc_front_body,
      out_shape=[jax.ShapeDtypeStruct((_N, _D), jnp.float32),
                 jax.ShapeDtypeStruct((_N, _D), jnp.float32)],
  )(x, g, b, wl, bl, wr, br)


def _tc_mid_body(op_ref, dp_ref, bias_ref, g_ref, b_ref, wl_ref, bl_ref,
                 wr_ref, br_ref, xl_ref, xr_ref):
  num = op_ref[0, :_N, :] + op_ref[1, :_N, :]
  den = jnp.sum(dp_ref[...], axis=0)[: _N]
  out = num / den[:, None] + bias_ref[...][None, :]
  h = jnp.maximum(out, 0.0)
  mu = jnp.mean(h, axis=0, keepdims=True)
  var = jnp.mean((h - mu) * (h - mu), axis=0, keepdims=True)
  h = (h - mu) * jax.lax.rsqrt(var + 1e-5) * g_ref[...][None, :] + b_ref[...][None, :]
  xl_ref[...] = jnp.dot(h, wl_ref[...], preferred_element_type=jnp.float32) + bl_ref[...][None, :]
  xr_ref[...] = jnp.dot(h, wr_ref[...], preferred_element_type=jnp.float32) + br_ref[...][None, :]


def _tc_mid(op, dp, bias, g, b, wl, bl, wr, br):
  return pl.pallas_call(
      _tc_mid_body,
      out_shape=[jax.ShapeDtypeStruct((_N, _D), jnp.float32),
                 jax.ShapeDtypeStruct((_N, _D), jnp.float32)],
  )(op, dp, bias, g, b, wl, bl, wr, br)


def _tc_final_body(op_ref, dp_ref, bias_ref, out_ref):
  num = op_ref[0, :_N, :] + op_ref[1, :_N, :]
  den = jnp.sum(dp_ref[...], axis=0)[: _N]
  out_ref[...] = num / den[:, None] + bias_ref[...][None, :]


def _tc_final(op, dp, bias):
  return pl.pallas_call(
      _tc_final_body,
      out_shape=jax.ShapeDtypeStruct((_N, _D), jnp.float32),
  )(op, dp, bias)


def _sc_logits_body(xl_hbm, xr_hbm, att_hbm, sd_hbm,
                    logits_hbm, mparts_hbm,
                    att_v, sdidx, xlr, xrr, lg, m_local, sem0, sem1):
  cid = lax.axis_index("c")
  sid = lax.axis_index("s")
  wid = sid * _NC + cid
  pltpu.sync_copy(att_hbm, att_v)
  _vfill(m_local, _NPAD // 16, _NEG, jnp.float32)

  def chunk_body(c, _):
    base = pl.multiple_of((wid * _CT + c) * _CHUNK, _CHUNK)
    pltpu.sync_copy(sd_hbm.at[wid * _CT + c], sdidx)
    cp0 = pltpu.async_copy(xl_hbm.at[sdidx.at[0]], xlr, sem0)
    cp1 = pltpu.async_copy(xr_hbm.at[sdidx.at[1]], xrr, sem1)
    cp0.wait()
    cp1.wait()

    io = lax.iota(jnp.int32, 16)
    for g in range(8):
      gx = pl.ds(g * 16, 16)

      def ebody(jj, acc, g=g):
        e = g * 16 + jj
        parts = []
        for j in range(8):
          jx = pl.ds(j * 16, 16)
          sval = xlr[e, jx] + xrr[e, jx]
          sval = jnp.maximum(sval, sval * 0.2)
          parts.append(sval * att_v[jx])
        while len(parts) > 1:
          parts = [a + b for a, b in zip(parts[::2], parts[1::2])]
        tot = jnp.sum(parts[0])
        return jnp.where(io == jj, tot, acc)

      lgv = lax.fori_loop(0, 16, ebody, jnp.zeros((16,), jnp.float32),
                          unroll=4)
      lg[gx] = lgv
      dstv = sdidx[1, gx]
      _seg_reduce_rmw(dstv, lgv, m_local, jnp.maximum)
    pltpu.sync_copy(lg, logits_hbm.at[pl.ds(base, _CHUNK)])
    return 0

  lax.fori_loop(0, _CT, chunk_body, 0)
  pltpu.sync_copy(m_local, mparts_hbm.at[wid])


def _sc_logits(xl, xr, att, sd):
  mesh = plsc.VectorSubcoreMesh(core_axis_name="c", subcore_axis_name="s")
  f = pl.kernel(
      _sc_logits_body,
      compiler_params=pltpu.CompilerParams(needs_layout_passes=False),
      out_type=[jax.ShapeDtypeStruct((_E2PAD,), jnp.float32),
                jax.ShapeDtypeStruct((_NW, _NPAD), jnp.float32)],
      mesh=mesh,
      scratch_types=[
          pltpu.VMEM((_D,), jnp.float32),
          pltpu.VMEM((2, _CHUNK), jnp.int32),
          pltpu.VMEM((_CHUNK, _D), jnp.float32),
          pltpu.VMEM((_CHUNK, _D), jnp.float32),
          pltpu.VMEM((_CHUNK,), jnp.float32),
          pltpu.VMEM((_NPAD,), jnp.float32),
          pltpu.SemaphoreType.DMA,
          pltpu.SemaphoreType.DMA,
      ],
  )
  return f(xl, xr, att, sd)


def _sc_aggregate_body(xl_hbm, sd_hbm, logits_hbm, mparts_hbm,
                       oparts_hbm, dparts_hbm,
                       sdidx, lg, rows, m_v, den_local, mstage, mloc,
                       shared, m_shared, sem0):
  cid = lax.axis_index("c")
  sid = lax.axis_index("s")
  wid = sid * _NC + cid
  stripe = _NPAD // _NS

  sbase = pl.multiple_of(sid * stripe, stripe)
  for r in range(2):
    cps = [pltpu.async_copy(mparts_hbm.at[r * _NS + t, pl.ds(sbase, stripe)],
                            mstage.at[t], sem0) for t in range(_NS)]
    for cp in cps:
      cp.wait()

    def cb(i, _, r=r):
      ix = pl.ds(pl.multiple_of(i * 16, 16), 16)
      parts = [mstage[t, ix] for t in range(_NS)]
      while len(parts) > 1:
        parts = [jnp.maximum(a, b) for a, b in zip(parts[::2], parts[1::2])]
      if r == 0:
        mloc[ix] = parts[0]
      else:
        mloc[ix] = jnp.maximum(mloc[ix], parts[0])
      return 0

    lax.fori_loop(0, stripe // 16, cb, 0)
  pltpu.sync_copy(mloc, m_shared.at[pl.ds(sbase, stripe)])

  _vfill(den_local, _NPAD // 16, 0.0, jnp.float32)

  def zrow(r, _):
    for j in range(8):
      rows[r, pl.ds(j * 16, 16)] = jnp.zeros((16,), jnp.float32)
    return 0

  lax.fori_loop(0, _CHUNK, zrow, 0)
  for k in range(stripe // _CHUNK):
    pltpu.sync_copy(rows, shared.at[pl.ds(sid * stripe + k * _CHUNK, _CHUNK)])
  plsc.subcore_barrier()
  pltpu.sync_copy(m_shared, m_v)

  def chunk_body(c, _):
    base = pl.multiple_of((wid * _CT + c) * _CHUNK, _CHUNK)
    pltpu.sync_copy(sd_hbm.at[wid * _CT + c], sdidx)
    pltpu.sync_copy(logits_hbm.at[pl.ds(base, _CHUNK)], lg)
    cp0 = pltpu.async_copy(xl_hbm.at[sdidx.at[0]], rows, sem0)
    cp0.wait()
    for g in range(8):
      gx = pl.ds(g * 16, 16)
      dstv = sdidx[1, gx]
      mg = plsc.load_gather(m_v, [dstv])
      w = jnp.exp(lg[gx] - mg)
      _seg_reduce_rmw(dstv, w, den_local, lax.add)
      for jj in range(16):
        e = g * 16 + jj
        ws = w[jj]
        for j in range(8):
          jx = pl.ds(j * 16, 16)
          rows[e, jx] = rows[e, jx] * ws

    pltpu.sync_copy(rows, shared.at[sdidx.at[1]], add=True)
    return 0

  lax.fori_loop(0, _CT, chunk_body, 0)
  plsc.subcore_barrier()
  pltpu.sync_copy(shared.at[pl.ds(sid * stripe, stripe)],
                  oparts_hbm.at[cid, pl.ds(sid * stripe, stripe)])
  pltpu.sync_copy(den_local, dparts_hbm.at[wid])


def _sc_aggregate(xl, sd, logits, mparts):
  mesh = plsc.VectorSubcoreMesh(core_axis_name="c", subcore_axis_name="s")
  f = pl.kernel(
      _sc_aggregate_body,
      compiler_params=pltpu.CompilerParams(needs_layout_passes=False),
      out_type=[jax.ShapeDtypeStruct((_NC, _NPAD, _D), jnp.float32),
                jax.ShapeDtypeStruct((_NW, _NPAD), jnp.float32)],
      mesh=mesh,
      scratch_types=[
          pltpu.VMEM((2, _CHUNK), jnp.int32),
          pltpu.VMEM((_CHUNK,), jnp.float32),
          pltpu.VMEM((_CHUNK, _D), jnp.float32),
          pltpu.VMEM((_NPAD,), jnp.float32),
          pltpu.VMEM((_NPAD,), jnp.float32),
          pltpu.VMEM((_NS, _NPAD // _NS), jnp.float32),
          pltpu.VMEM((_NPAD // _NS,), jnp.float32),
          pltpu.VMEM_SHARED((_NPAD, _D), jnp.float32),
          pltpu.VMEM_SHARED((_NPAD,), jnp.float32),
          pltpu.SemaphoreType.DMA,
      ],
  )
  return f(xl, sd, logits, mparts)


def _edge_stage(xl, xr, att, sd):
  xlp = jnp.pad(xl, ((0, _NPAD - _N), (0, 0)))
  xrp = jnp.pad(xr, ((0, _NPAD - _N), (0, 0)))
  logits, mparts = _sc_logits(xlp, xrp, att.reshape(_D), sd)
  return _sc_aggregate(xlp, sd, logits, mparts)


def kernel(x, edge_index, edge_weight, bn1_g, bn1_b, Wl1, bl1, Wr1, br1,
           att1, bias1, bn2_g, bn2_b, Wl2, bl2, Wr2, br2, att2, bias2):
  del edge_weight
  src = edge_index[0]
  dst = edge_index[1]
  loop = jnp.arange(_N, dtype=src.dtype)
  pad = jnp.full((_E2RD - _E2,), _NPAD - 1, src.dtype)
  s2 = jnp.concatenate([src, loop, pad])
  d2 = jnp.concatenate([dst, loop, pad])
  sd = jnp.stack([s2.reshape(-1, _CHUNK), d2.reshape(-1, _CHUNK)], axis=1)

  xl1, xr1 = _tc_front(x, bn1_g, bn1_b, Wl1, bl1, Wr1, br1)
  op1, dp1 = _edge_stage(xl1, xr1, att1, sd)
  xl2, xr2 = _tc_mid(op1, dp1, bias1, bn2_g, bn2_b, Wl2, bl2, Wr2, br2)
  op2, dp2 = _edge_stage(xl2, xr2, att2, sd)
  return _tc_final(op2, dp2, bias2)
